# Initial kernel scaffold; baseline (speedup 1.0000x reference)
import jax
import jax.numpy as jnp
from jax.experimental import pallas as pl


def kernel(placeholder_inputs, origin_embeddings, codebook, W_t, W_se):
    raise NotImplementedError("write your pallas kernel here")


# speedup vs baseline: 14.3646x; 14.3646x over previous
import functools

import numpy as np
import jax
import jax.numpy as jnp
from jax import lax
from jax.experimental import pallas as pl
from jax.experimental.pallas import tpu as pltpu
from jax.experimental.pallas import tpu_sc as plsc

_F = 26
_TOP = 6
_E = 16
_OUT = 32
_B = 4096
_NBKT = 1000000

_PAIR_LIST = [(i, j) for i in range(_F) for j in range(i + 1, _F)
              if (i < _TOP or j < _TOP)]
_NI = len(_PAIR_LIST)
_PA_IDX = np.array([p[0] for p in _PAIR_LIST], dtype=np.int32)
_PB_IDX = np.array([p[1] for p in _PAIR_LIST], dtype=np.int32)
_ROW_OFF = [0, 25, 49, 72, 94, 115]

_NC = 2
_NS = 16
_NW = _NC * _NS
_BPW = _B // _NW
_SLOTS = _BPW * _NI
_CH = 128
_NCHUNK = _SLOTS // _CH


def _sc_gather(pa_flat, pb_flat, codebook):
    mesh = plsc.VectorSubcoreMesh(core_axis_name="c", subcore_axis_name="s")

    @functools.partial(
        pl.kernel,
        out_type=(jax.ShapeDtypeStruct((_B * _NI, _E), jnp.float32),
                  jax.ShapeDtypeStruct((_B * _NI, _E), jnp.float32)),
        mesh=mesh,
        compiler_params=pltpu.CompilerParams(use_tc_tiling_on_sc=False),
        scratch_types=[
            pltpu.VMEM((_SLOTS,), jnp.int32),
            pltpu.VMEM((_SLOTS,), jnp.int32),
            pltpu.VMEM((_CH,), jnp.int32),
            pltpu.VMEM((_CH,), jnp.int32),
            pltpu.VMEM((_CH, _E), jnp.float32),
            pltpu.VMEM((_CH, _E), jnp.float32),
            pltpu.SemaphoreType.DMA,
            pltpu.SemaphoreType.DMA,
        ],
    )
    def k(pa_hbm, pb_hbm, cb_hbm, g0_hbm, g1_hbm,
          pa_v, pb_v, idx0_v, idx1_v, r0_v, r1_v, sem0, sem1):
        wid = lax.axis_index("s") * _NC + lax.axis_index("c")
        base = wid * _SLOTS
        pltpu.sync_copy(pa_hbm.at[pl.ds(base, _SLOTS)], pa_v)
        pltpu.sync_copy(pb_hbm.at[pl.ds(base, _SLOTS)], pb_v)

        def body(c, carry):
            off = c * _CH
            for v in range(_CH // 16):
                s = off + v * 16
                a = pa_v[pl.ds(s, 16)].astype(jnp.uint32)
                b = pb_v[pl.ds(s, 16)].astype(jnp.uint32)
                tok = a * jnp.uint32(2654435761) + b
                b0 = (tok * jnp.uint32(7744) + jnp.uint32(1822)) % jnp.uint32(_NBKT)
                b1 = (tok * jnp.uint32(423) + jnp.uint32(6649)) % jnp.uint32(_NBKT)
                idx0_v[pl.ds(v * 16, 16)] = b0.astype(jnp.int32)
                idx1_v[pl.ds(v * 16, 16)] = b1.astype(jnp.int32)
            cp0 = pltpu.async_copy(cb_hbm.at[idx0_v], r0_v, sem0)
            cp1 = pltpu.async_copy(cb_hbm.at[idx1_v], r1_v, sem1)
            cp0.wait()
            cp1.wait()
            pltpu.sync_copy(r0_v, g0_hbm.at[pl.ds(base + off, _CH)])
            pltpu.sync_copy(r1_v, g1_hbm.at[pl.ds(base + off, _CH)])
            return carry

        lax.fori_loop(0, _NCHUNK, body, 0)

    return k(pa_flat, pb_flat, codebook)


def _aggregate(wm):
    parts = []
    for i in range(_TOP):
        s = jnp.sum(wm[:, _ROW_OFF[i]:_ROW_OFF[i] + (_F - 1 - i), :], axis=1)
        for t in range(i):
            s = s + wm[:, _ROW_OFF[t] + i - t - 1, :]
        parts.append(s[:, None, :])
    nontop = wm[:, _ROW_OFF[0] + _TOP - 1:_ROW_OFF[0] + _F - 1, :]
    for t in range(1, _TOP):
        lo = _ROW_OFF[t] + _TOP - 1 - t
        nontop = nontop + wm[:, lo:lo + (_F - _TOP), :]
    parts.append(nontop)
    return jnp.concatenate(parts, axis=1)


_TB = 64


def _tc_body(g0_ref, g1_ref, z_ref, wse_ref, wt0_ref, wt1_ref, o_ref):
    w = jnp.dot(z_ref[...], wse_ref[...], preferred_element_type=jnp.float32)
    w3 = w[:, :, None]
    q0 = _aggregate(g0_ref[...] * w3)
    q1 = _aggregate(g1_ref[...] * w3)
    out = (jnp.dot(q0.reshape(_TB * _F, _E), wt0_ref[...],
                   preferred_element_type=jnp.float32)
           + jnp.dot(q1.reshape(_TB * _F, _E), wt1_ref[...],
                     preferred_element_type=jnp.float32))
    o_ref[...] = out.reshape(_TB, _F, _OUT)


def _tc_dense(g0, g1, z, wse, wt0, wt1):
    grid = _B // _TB
    return pl.pallas_call(
        _tc_body,
        grid=(grid,),
        in_specs=[
            pl.BlockSpec((_TB, _NI, _E), lambda i: (i, 0, 0)),
            pl.BlockSpec((_TB, _NI, _E), lambda i: (i, 0, 0)),
            pl.BlockSpec((_TB, _F * _E), lambda i: (i, 0)),
            pl.BlockSpec((_F * _E, _NI), lambda i: (0, 0)),
            pl.BlockSpec((_E, _OUT), lambda i: (0, 0)),
            pl.BlockSpec((_E, _OUT), lambda i: (0, 0)),
        ],
        out_specs=pl.BlockSpec((_TB, _F, _OUT), lambda i: (i, 0, 0)),
        out_shape=jax.ShapeDtypeStruct((_B, _F, _OUT), jnp.float32),
    )(g0, g1, z, wse, wt0, wt1)


def kernel(placeholder_inputs, origin_embeddings, codebook, W_t, W_se):
    ids = placeholder_inputs
    pa = jnp.take(ids, jnp.asarray(_PA_IDX), axis=1).reshape(-1)
    pb = jnp.take(ids, jnp.asarray(_PB_IDX), axis=1).reshape(-1)
    g0, g1 = _sc_gather(pa, pb, codebook)
    z = origin_embeddings.reshape(_B, _F * _E)
    out = _tc_dense(g0.reshape(_B, _NI, _E), g1.reshape(_B, _NI, _E),
                    z, W_se, W_t[:_E], W_t[_E:])
    return out


# speedup vs baseline: 26.7430x; 1.8617x over previous
import functools

import numpy as np
import jax
import jax.numpy as jnp
from jax import lax
from jax.experimental import pallas as pl
from jax.experimental.pallas import tpu as pltpu
from jax.experimental.pallas import tpu_sc as plsc

_F = 26
_TOP = 6
_E = 16
_OUT = 32
_B = 4096
_NBKT = 1000000

_PAIR_LIST = [(i, j) for i in range(_F) for j in range(i + 1, _F)
              if (i < _TOP or j < _TOP)]
_NI = len(_PAIR_LIST)
_NIP = 144
_FI = [p[0] for p in _PAIR_LIST] + [0] * (_NIP - _NI)
_FJ = [p[1] for p in _PAIR_LIST] + [0] * (_NIP - _NI)
_PA_IDX = np.array(_FI, dtype=np.int32)
_PB_IDX = np.array(_FJ, dtype=np.int32)

_ACC_W = _F * 2 * _E

_NC = 2
_NS = 16
_NW = _NC * _NS
_RPW = _B // _NW
_GRP = 8
_NGRP = _RPW // _GRP
_NSTR = 3
_SW = _NIP // _NSTR


def _sc_gather_agg(pa_flat, pb_flat, w_flat, codebook):
    mesh = plsc.VectorSubcoreMesh(core_axis_name="c", subcore_axis_name="s")
    stage = _GRP * _NIP

    @functools.partial(
        pl.kernel,
        out_type=jax.ShapeDtypeStruct((_B * _ACC_W,), jnp.float32),
        mesh=mesh,
        compiler_params=pltpu.CompilerParams(use_tc_tiling_on_sc=False),
        scratch_types=[
            pltpu.VMEM((_RPW * _ACC_W,), jnp.float32),
            pltpu.VMEM((stage,), jnp.int32),
            pltpu.VMEM((stage,), jnp.int32),
            pltpu.VMEM((stage,), jnp.float32),
            pltpu.VMEM((_NSTR, _SW), jnp.int32),
            pltpu.VMEM((_NSTR, _SW), jnp.int32),
            pltpu.VMEM((_NIP, _E), jnp.float32),
            pltpu.VMEM((_NIP, _E), jnp.float32),
            pltpu.SemaphoreType.DMA,
        ],
    )
    def k(pa_hbm, pb_hbm, w_hbm, cb_hbm, acc_hbm,
          acc_v, pa_v, pb_v, w_v, idx0_v, idx1_v, r0_v, r1_v, sem):
        wid = lax.axis_index("s") * _NC + lax.axis_index("c")
        slot_base = wid * _RPW * _NIP

        def zero(i, carry):
            acc_v[pl.ds(i * 16, 16)] = jnp.zeros((16,), jnp.float32)
            return carry
        lax.fori_loop(0, _RPW * _ACC_W // 16, zero, 0)

        def group(g, carry):
            goff = slot_base + g * stage
            pltpu.sync_copy(pa_hbm.at[pl.ds(goff, stage)], pa_v)
            pltpu.sync_copy(pb_hbm.at[pl.ds(goff, stage)], pb_v)
            pltpu.sync_copy(w_hbm.at[pl.ds(goff, stage)], w_v)

            def row(r, carry2):
                roff = r * _NIP
                for v in range(_NIP // 16):
                    a = pa_v[pl.ds(roff + v * 16, 16)].astype(jnp.uint32)
                    b = pb_v[pl.ds(roff + v * 16, 16)].astype(jnp.uint32)
                    tok = a * jnp.uint32(2654435761) + b
                    b0 = (tok * jnp.uint32(7744) + jnp.uint32(1822)) % jnp.uint32(_NBKT)
                    b1 = (tok * jnp.uint32(423) + jnp.uint32(6649)) % jnp.uint32(_NBKT)
                    st, sc = divmod(v * 16, _SW)
                    idx0_v[st, pl.ds(sc, 16)] = b0.astype(jnp.int32)
                    idx1_v[st, pl.ds(sc, 16)] = b1.astype(jnp.int32)
                cps = []
                for st in range(_NSTR):
                    cps.append(pltpu.async_copy(
                        cb_hbm.at[idx0_v.at[st]],
                        r0_v.at[pl.ds(st * _SW, _SW)], sem))
                    cps.append(pltpu.async_copy(
                        cb_hbm.at[idx1_v.at[st]],
                        r1_v.at[pl.ds(st * _SW, _SW)], sem))
                for cp in cps:
                    cp.wait()
                abase = (g * _GRP + r) * _ACC_W
                for v in range(_NIP // 16):
                    wv = w_v[pl.ds(roff + v * 16, 16)]
                    for sl in range(16):
                        s = v * 16 + sl
                        ws = jnp.broadcast_to(wv[sl], (16,))
                        wr0 = r0_v[s] * ws
                        wr1 = r1_v[s] * ws
                        ci = abase + _FI[s] * 2 * _E
                        cj = abase + _FJ[s] * 2 * _E
                        plsc.addupdate(acc_v.at[pl.ds(ci, 16)], wr0)
                        plsc.addupdate(acc_v.at[pl.ds(ci + 16, 16)], wr1)
                        plsc.addupdate(acc_v.at[pl.ds(cj, 16)], wr0)
                        plsc.addupdate(acc_v.at[pl.ds(cj + 16, 16)], wr1)
                return carry2
            lax.fori_loop(0, _GRP, row, 0)
            return carry
        lax.fori_loop(0, _NGRP, group, 0)
        pltpu.sync_copy(acc_v, acc_hbm.at[pl.ds(wid * _RPW * _ACC_W, _RPW * _ACC_W)])

    return k(pa_flat, pb_flat, w_flat, codebook)


_TB1 = 512


def _w_body(z_ref, wse_ref, o_ref):
    o_ref[...] = jnp.dot(z_ref[...], wse_ref[...],
                         preferred_element_type=jnp.float32)


def _w_matmul(z, wse_pad):
    return pl.pallas_call(
        _w_body,
        grid=(_B // _TB1,),
        in_specs=[
            pl.BlockSpec((_TB1, _F * _E), lambda i: (i, 0)),
            pl.BlockSpec((_F * _E, _NIP), lambda i: (0, 0)),
        ],
        out_specs=pl.BlockSpec((_TB1, _NIP), lambda i: (i, 0)),
        out_shape=jax.ShapeDtypeStruct((_B, _NIP), jnp.float32),
    )(z, wse_pad)


_TB2 = 512


def _out_body(a_ref, wbd_ref, o_ref):
    o_ref[...] = jnp.dot(a_ref[...], wbd_ref[...],
                         preferred_element_type=jnp.float32)


def _out_matmul(acc2, wbd):
    return pl.pallas_call(
        _out_body,
        grid=(_B // _TB2,),
        in_specs=[
            pl.BlockSpec((_TB2, _ACC_W), lambda i: (i, 0)),
            pl.BlockSpec((_ACC_W, _ACC_W), lambda i: (0, 0)),
        ],
        out_specs=pl.BlockSpec((_TB2, _ACC_W), lambda i: (i, 0)),
        out_shape=jax.ShapeDtypeStruct((_B, _ACC_W), jnp.float32),
    )(acc2, wbd)


def kernel(placeholder_inputs, origin_embeddings, codebook, W_t, W_se):
    ids = placeholder_inputs
    pa = jnp.take(ids, jnp.asarray(_PA_IDX), axis=1).reshape(-1)
    pb = jnp.take(ids, jnp.asarray(_PB_IDX), axis=1).reshape(-1)
    z = origin_embeddings.reshape(_B, _F * _E)
    wse_pad = jnp.pad(W_se, ((0, 0), (0, _NIP - _NI)))
    w = _w_matmul(z, wse_pad).reshape(-1)
    acc = _sc_gather_agg(pa, pb, w, codebook).reshape(_B, _ACC_W)
    wbd = jnp.kron(jnp.eye(_F, dtype=jnp.float32), W_t)
    out = _out_matmul(acc, wbd)
    return out.reshape(_B, _F, _OUT)


# speedup vs baseline: 29.9609x; 1.1203x over previous
import functools

import numpy as np
import jax
import jax.numpy as jnp
from jax import lax
from jax.experimental import pallas as pl
from jax.experimental.pallas import tpu as pltpu
from jax.experimental.pallas import tpu_sc as plsc

_F = 26
_TOP = 6
_E = 16
_OUT = 32
_B = 4096
_NBKT = 1000000

_PAIR_LIST = [(i, j) for i in range(_F) for j in range(i + 1, _F)
              if (i < _TOP or j < _TOP)]
_NI = len(_PAIR_LIST)
_NIP = 144
_FI = [p[0] for p in _PAIR_LIST] + [0] * (_NIP - _NI)
_FJ = [p[1] for p in _PAIR_LIST] + [0] * (_NIP - _NI)
_PA_IDX = np.array(_FI, dtype=np.int32)
_PB_IDX = np.array(_FJ, dtype=np.int32)

_ACC_W = _F * 2 * _E

_NC = 2
_NS = 16
_NW = _NC * _NS
_RPW = _B // _NW
_GRP = 8
_NGRP = _RPW // _GRP
_NSTR = 3
_SW = _NIP // _NSTR


def _sc_gather_agg(pa_flat, pb_flat, w_flat, codebook):
    mesh = plsc.VectorSubcoreMesh(core_axis_name="c", subcore_axis_name="s")
    stage = _GRP * _NIP

    @functools.partial(
        pl.kernel,
        out_type=jax.ShapeDtypeStruct((_B * _ACC_W,), jnp.float32),
        mesh=mesh,
        compiler_params=pltpu.CompilerParams(use_tc_tiling_on_sc=False),
        scratch_types=[
            pltpu.VMEM((_RPW * _ACC_W,), jnp.float32),
            pltpu.VMEM((stage,), jnp.int32),
            pltpu.VMEM((stage,), jnp.int32),
            pltpu.VMEM((stage,), jnp.float32),
            pltpu.VMEM((_NSTR, _SW), jnp.int32),
            pltpu.VMEM((_NSTR, _SW), jnp.int32),
            pltpu.VMEM((_NIP, _E), jnp.float32),
            pltpu.VMEM((_NIP, _E), jnp.float32),
            pltpu.SemaphoreType.DMA,
        ],
    )
    def k(pa_hbm, pb_hbm, w_hbm, cb_hbm, acc_hbm,
          acc_v, pa_v, pb_v, w_v, idx0_v, idx1_v, r0_v, r1_v, sem):
        wid = lax.axis_index("s") * _NC + lax.axis_index("c")
        slot_base = wid * _RPW * _NIP

        def zero(i, carry):
            acc_v[pl.ds(i * 16, 16)] = jnp.zeros((16,), jnp.float32)
            return carry
        lax.fori_loop(0, _RPW * _ACC_W // 16, zero, 0)

        def group(g, carry):
            goff = slot_base + g * stage
            pltpu.sync_copy(pa_hbm.at[pl.ds(goff, stage)], pa_v)
            pltpu.sync_copy(pb_hbm.at[pl.ds(goff, stage)], pb_v)
            pltpu.sync_copy(w_hbm.at[pl.ds(goff, stage)], w_v)

            def row(r, carry2):
                roff = r * _NIP
                for v in range(_NIP // 16):
                    a = pa_v[pl.ds(roff + v * 16, 16)].astype(jnp.uint32)
                    b = pb_v[pl.ds(roff + v * 16, 16)].astype(jnp.uint32)
                    tok = a * jnp.uint32(2654435761) + b
                    b0 = (tok * jnp.uint32(7744) + jnp.uint32(1822)) % jnp.uint32(_NBKT)
                    b1 = (tok * jnp.uint32(423) + jnp.uint32(6649)) % jnp.uint32(_NBKT)
                    st, sc = divmod(v * 16, _SW)
                    idx0_v[st, pl.ds(sc, 16)] = b0.astype(jnp.int32)
                    idx1_v[st, pl.ds(sc, 16)] = b1.astype(jnp.int32)
                cps = []
                for st in range(_NSTR):
                    cps.append(pltpu.async_copy(
                        cb_hbm.at[idx0_v.at[st]],
                        r0_v.at[pl.ds(st * _SW, _SW)], sem))
                    cps.append(pltpu.async_copy(
                        cb_hbm.at[idx1_v.at[st]],
                        r1_v.at[pl.ds(st * _SW, _SW)], sem))
                for cp in cps:
                    cp.wait()
                abase = (g * _GRP + r) * _ACC_W
                for v in range(_NIP // 16):
                    wv = w_v[pl.ds(roff + v * 16, 16)]
                    for sl in range(16):
                        s = v * 16 + sl
                        ws = jnp.broadcast_to(wv[sl], (16,))
                        wr0 = r0_v[s] * ws
                        wr1 = r1_v[s] * ws
                        ci = abase + _FI[s] * 2 * _E
                        cj = abase + _FJ[s] * 2 * _E
                        plsc.addupdate(acc_v.at[pl.ds(ci, 16)], wr0)
                        plsc.addupdate(acc_v.at[pl.ds(ci + 16, 16)], wr1)
                        plsc.addupdate(acc_v.at[pl.ds(cj, 16)], wr0)
                        plsc.addupdate(acc_v.at[pl.ds(cj + 16, 16)], wr1)
                return carry2
            lax.fori_loop(0, _GRP, row, 0)
            return carry
        lax.fori_loop(0, _NGRP, group, 0)
        pltpu.sync_copy(acc_v, acc_hbm.at[pl.ds(wid * _RPW * _ACC_W, _RPW * _ACC_W)])

    return k(pa_flat, pb_flat, w_flat, codebook)


_TCH = 15360
_NBLK = -(-_NBKT // _TCH)
_NBKT_PAD = _NBLK * _TCH


def _relayout_body(ct_ref, o_ref):
    xt = ct_ref[...].T
    xt3 = xt.reshape(_TCH // 8, 8, _E)
    o_ref[...] = jnp.concatenate([xt3[:, k, :] for k in range(8)], axis=-1)


def _cb_relayout(cb_t):
    return pl.pallas_call(
        _relayout_body,
        grid=(_NBLK,),
        in_specs=[pl.BlockSpec((_E, _TCH), lambda i: (0, i))],
        out_specs=pl.BlockSpec((_TCH * _E // 128, 128), lambda i: (i, 0)),
        out_shape=jax.ShapeDtypeStruct((_NBKT_PAD * _E // 128, 128),
                                       jnp.float32),
    )(cb_t)


_TB1 = 512


def _w_body(z_ref, wse_ref, o_ref):
    o_ref[...] = jnp.dot(z_ref[...], wse_ref[...],
                         preferred_element_type=jnp.float32)


def _w_matmul(z, wse_pad):
    return pl.pallas_call(
        _w_body,
        grid=(_B // _TB1,),
        in_specs=[
            pl.BlockSpec((_TB1, _F * _E), lambda i: (i, 0)),
            pl.BlockSpec((_F * _E, _NIP), lambda i: (0, 0)),
        ],
        out_specs=pl.BlockSpec((_TB1, _NIP), lambda i: (i, 0)),
        out_shape=jax.ShapeDtypeStruct((_B, _NIP), jnp.float32),
    )(z, wse_pad)


_TB2 = 512


def _out_body(a_ref, wbd_ref, o_ref):
    o_ref[...] = jnp.dot(a_ref[...], wbd_ref[...],
                         preferred_element_type=jnp.float32)


def _out_matmul(acc2, wbd):
    return pl.pallas_call(
        _out_body,
        grid=(_B // _TB2,),
        in_specs=[
            pl.BlockSpec((_TB2, _ACC_W), lambda i: (i, 0)),
            pl.BlockSpec((_ACC_W, _ACC_W), lambda i: (0, 0)),
        ],
        out_specs=pl.BlockSpec((_TB2, _ACC_W), lambda i: (i, 0)),
        out_shape=jax.ShapeDtypeStruct((_B, _ACC_W), jnp.float32),
    )(acc2, wbd)


def kernel(placeholder_inputs, origin_embeddings, codebook, W_t, W_se):
    ids = placeholder_inputs
    pa = jnp.take(ids, jnp.asarray(_PA_IDX), axis=1).reshape(-1)
    pb = jnp.take(ids, jnp.asarray(_PB_IDX), axis=1).reshape(-1)
    z = origin_embeddings.reshape(_B, _F * _E)
    wse_pad = jnp.pad(W_se, ((0, 0), (0, _NIP - _NI)))
    w = _w_matmul(z, wse_pad).reshape(-1)
    cb_lin = _cb_relayout(codebook.T).reshape(_NBKT_PAD, _E)
    acc = _sc_gather_agg(pa, pb, w, cb_lin).reshape(_B, _ACC_W)
    wbd = jnp.kron(jnp.eye(_F, dtype=jnp.float32), W_t)
    out = _out_matmul(acc, wbd)
    return out.reshape(_B, _F, _OUT)


# speedup vs baseline: 31.8257x; 1.0622x over previous
import functools

import numpy as np
import jax
import jax.numpy as jnp
from jax import lax
from jax.experimental import pallas as pl
from jax.experimental.pallas import tpu as pltpu
from jax.experimental.pallas import tpu_sc as plsc

_F = 26
_TOP = 6
_E = 16
_OUT = 32
_B = 4096
_NBKT = 1000000

_PAIR_LIST = [(i, j) for i in range(_F) for j in range(i + 1, _F)
              if (i < _TOP or j < _TOP)]
_NI = len(_PAIR_LIST)
_NIP = 144
_FI = [p[0] for p in _PAIR_LIST] + [0] * (_NIP - _NI)
_FJ = [p[1] for p in _PAIR_LIST] + [0] * (_NIP - _NI)
_PA_IDX = np.array(_FI, dtype=np.int32)
_PB_IDX = np.array(_FJ, dtype=np.int32)

_ACC_W = _F * 2 * _E

_NC = 2
_NS = 16
_NW = _NC * _NS
_RPW = _B // _NW
_GRP = 8
_NGRP = _RPW // _GRP
_NSTR = 3
_SW = _NIP // _NSTR


def _sc_gather_agg(pa_flat, pb_flat, w_flat, codebook):
    mesh = plsc.VectorSubcoreMesh(core_axis_name="c", subcore_axis_name="s")
    stage = _GRP * _NIP

    @functools.partial(
        pl.kernel,
        out_type=jax.ShapeDtypeStruct((_B * _ACC_W,), jnp.float32),
        mesh=mesh,
        compiler_params=pltpu.CompilerParams(use_tc_tiling_on_sc=False),
        scratch_types=[
            pltpu.VMEM((_RPW * _ACC_W,), jnp.float32),
            pltpu.VMEM((stage,), jnp.int32),
            pltpu.VMEM((stage,), jnp.int32),
            pltpu.VMEM((stage,), jnp.float32),
            pltpu.VMEM((_NSTR, _SW), jnp.int32),
            pltpu.VMEM((_NSTR, _SW), jnp.int32),
            pltpu.VMEM((_NIP, _E), jnp.float32),
            pltpu.VMEM((_NIP, _E), jnp.float32),
            pltpu.SemaphoreType.DMA,
        ],
    )
    def k(pa_hbm, pb_hbm, w_hbm, cb_hbm, acc_hbm,
          acc_v, pa_v, pb_v, w_v, idx0_v, idx1_v, r0_v, r1_v, sem):
        wid = lax.axis_index("s") * _NC + lax.axis_index("c")
        slot_base = wid * _RPW * _NIP

        def zero(i, carry):
            acc_v[pl.ds(i * 16, 16)] = jnp.zeros((16,), jnp.float32)
            return carry
        lax.fori_loop(0, _RPW * _ACC_W // 16, zero, 0)

        def group(g, carry):
            goff = slot_base + g * stage
            pltpu.sync_copy(pa_hbm.at[pl.ds(goff, stage)], pa_v)
            pltpu.sync_copy(pb_hbm.at[pl.ds(goff, stage)], pb_v)
            pltpu.sync_copy(w_hbm.at[pl.ds(goff, stage)], w_v)

            def row(r, carry2):
                roff = r * _NIP
                for v in range(_NIP // 16):
                    a = pa_v[pl.ds(roff + v * 16, 16)].astype(jnp.uint32)
                    b = pb_v[pl.ds(roff + v * 16, 16)].astype(jnp.uint32)
                    tok = a * jnp.uint32(2654435761) + b
                    b0 = (tok * jnp.uint32(7744) + jnp.uint32(1822)) % jnp.uint32(_NBKT)
                    b1 = (tok * jnp.uint32(423) + jnp.uint32(6649)) % jnp.uint32(_NBKT)
                    st, sc = divmod(v * 16, _SW)
                    idx0_v[st, pl.ds(sc, 16)] = b0.astype(jnp.int32)
                    idx1_v[st, pl.ds(sc, 16)] = b1.astype(jnp.int32)
                cps = []
                for st in range(_NSTR):
                    cps.append(pltpu.async_copy(
                        cb_hbm.at[idx0_v.at[st]],
                        r0_v.at[pl.ds(st * _SW, _SW)], sem))
                    cps.append(pltpu.async_copy(
                        cb_hbm.at[idx1_v.at[st]],
                        r1_v.at[pl.ds(st * _SW, _SW)], sem))
                for cp in cps:
                    cp.wait()
                abase = (g * _GRP + r) * _ACC_W
                for v in range(_NIP // 16):
                    wv = w_v[pl.ds(roff + v * 16, 16)]
                    for sl in range(16):
                        s = v * 16 + sl
                        ws = jnp.broadcast_to(wv[sl], (16,))
                        wr0 = r0_v[s] * ws
                        wr1 = r1_v[s] * ws
                        ci = abase + _FI[s] * 2 * _E
                        cj = abase + _FJ[s] * 2 * _E
                        plsc.addupdate(acc_v.at[pl.ds(ci, 16)], wr0)
                        plsc.addupdate(acc_v.at[pl.ds(ci + 16, 16)], wr1)
                        plsc.addupdate(acc_v.at[pl.ds(cj, 16)], wr0)
                        plsc.addupdate(acc_v.at[pl.ds(cj + 16, 16)], wr1)
                return carry2
            lax.fori_loop(0, _GRP, row, 0)
            return carry
        lax.fori_loop(0, _NGRP, group, 0)
        pltpu.sync_copy(acc_v, acc_hbm.at[pl.ds(wid * _RPW * _ACC_W, _RPW * _ACC_W)])

    return k(pa_flat, pb_flat, w_flat, codebook)


_TCH = 15360
_NBLK = -(-_NBKT // _TCH)
_NBKT_PAD = _NBLK * _TCH


def _relayout_body(ct_ref, o_ref):
    xt = ct_ref[...].T
    xt3 = xt.reshape(_TCH // 8, 8, _E)
    for k in range(8):
        o_ref[:, k * _E:(k + 1) * _E] = xt3[:, k, :]


def _cb_relayout(cb_t):
    return pl.pallas_call(
        _relayout_body,
        grid=(_NBLK,),
        in_specs=[pl.BlockSpec((_E, _TCH), lambda i: (0, i))],
        out_specs=pl.BlockSpec((_TCH * _E // 128, 128), lambda i: (i, 0)),
        out_shape=jax.ShapeDtypeStruct((_NBKT_PAD * _E // 128, 128),
                                       jnp.float32),
    )(cb_t)


_TB1 = 512


def _w_body(z_ref, wse_ref, o_ref):
    o_ref[...] = jnp.dot(z_ref[...], wse_ref[...],
                         preferred_element_type=jnp.float32)


def _w_matmul(z, wse_pad):
    return pl.pallas_call(
        _w_body,
        grid=(_B // _TB1,),
        in_specs=[
            pl.BlockSpec((_TB1, _F * _E), lambda i: (i, 0)),
            pl.BlockSpec((_F * _E, _NIP), lambda i: (0, 0)),
        ],
        out_specs=pl.BlockSpec((_TB1, _NIP), lambda i: (i, 0)),
        out_shape=jax.ShapeDtypeStruct((_B, _NIP), jnp.float32),
    )(z, wse_pad)


_TB2 = 512


def _out_body(a_ref, wbd_ref, o_ref):
    o_ref[...] = jnp.dot(a_ref[...], wbd_ref[...],
                         preferred_element_type=jnp.float32)


def _out_matmul(acc2, wbd):
    return pl.pallas_call(
        _out_body,
        grid=(_B // _TB2,),
        in_specs=[
            pl.BlockSpec((_TB2, _ACC_W), lambda i: (i, 0)),
            pl.BlockSpec((_ACC_W, _ACC_W), lambda i: (0, 0)),
        ],
        out_specs=pl.BlockSpec((_TB2, _ACC_W), lambda i: (i, 0)),
        out_shape=jax.ShapeDtypeStruct((_B, _ACC_W), jnp.float32),
    )(acc2, wbd)


def kernel(placeholder_inputs, origin_embeddings, codebook, W_t, W_se):
    ids = placeholder_inputs
    pa = jnp.take(ids, jnp.asarray(_PA_IDX), axis=1).reshape(-1)
    pb = jnp.take(ids, jnp.asarray(_PB_IDX), axis=1).reshape(-1)
    z = origin_embeddings.reshape(_B, _F * _E)
    wse_pad = jnp.pad(W_se, ((0, 0), (0, _NIP - _NI)))
    w = _w_matmul(z, wse_pad).reshape(-1)
    cb_lin = _cb_relayout(codebook.T).reshape(_NBKT_PAD, _E)
    acc = _sc_gather_agg(pa, pb, w, cb_lin).reshape(_B, _ACC_W)
    wbd = jnp.kron(jnp.eye(_F, dtype=jnp.float32), W_t)
    out = _out_matmul(acc, wbd)
    return out.reshape(_B, _F, _OUT)


# speedup vs baseline: 34.7974x; 1.0934x over previous
import functools

import numpy as np
import jax
import jax.numpy as jnp
from jax import lax
from jax.experimental import pallas as pl
from jax.experimental.pallas import tpu as pltpu
from jax.experimental.pallas import tpu_sc as plsc

_F = 26
_TOP = 6
_E = 16
_OUT = 32
_B = 4096
_NBKT = 1000000

_PAIR_LIST = [(i, j) for i in range(_F) for j in range(i + 1, _F)
              if (i < _TOP or j < _TOP)]
_NI = len(_PAIR_LIST)
_NIP = 144
_FI = [p[0] for p in _PAIR_LIST] + [0] * (_NIP - _NI)
_FJ = [p[1] for p in _PAIR_LIST] + [0] * (_NIP - _NI)
_PA_IDX = np.array(_FI, dtype=np.int32)
_PB_IDX = np.array(_FJ, dtype=np.int32)

_ACC_W = _F * 2 * _E

_NC = 2
_NS = 16
_NW = _NC * _NS
_RPW = _B // _NW
_PASS = 64
_NSTR = 3
_SW = _NIP // _NSTR

_TCH = 15360
_NBLK = -(-_NBKT // _TCH)
_NBKT_PAD = _NBLK * _TCH


def _relayout_body(ct_ref, o_ref):
    xt = ct_ref[...].T
    xt3 = xt.reshape(_TCH // 8, 8, _E)
    for k in range(8):
        o_ref[:, k * _E:(k + 1) * _E] = xt3[:, k, :]


def _cb_relayout(cb_t):
    return pl.pallas_call(
        _relayout_body,
        grid=(_NBLK,),
        in_specs=[pl.BlockSpec((_E, _TCH), lambda i: (0, i))],
        out_specs=pl.BlockSpec((_TCH * _E // 128, 128), lambda i: (i, 0)),
        out_shape=jax.ShapeDtypeStruct((_NBKT_PAD * _E // 128, 128),
                                       jnp.float32),
    )(cb_t)


_TB1 = 512


def _wh_body(z_ref, wse_ref, pa_ref, pb_ref, w_ref, b0_ref, b1_ref):
    w_ref[...] = jnp.dot(z_ref[...], wse_ref[...],
                         preferred_element_type=jnp.float32)
    a = pa_ref[...].astype(jnp.uint32)
    b = pb_ref[...].astype(jnp.uint32)
    tok = a * jnp.uint32(2654435761) + b
    b0_ref[...] = ((tok * jnp.uint32(7744) + jnp.uint32(1822))
                   % jnp.uint32(_NBKT)).astype(jnp.int32)
    b1_ref[...] = ((tok * jnp.uint32(423) + jnp.uint32(6649))
                   % jnp.uint32(_NBKT)).astype(jnp.int32)


def _w_and_hash(z, wse_pad, pa2, pb2):
    return pl.pallas_call(
        _wh_body,
        grid=(_B // _TB1,),
        in_specs=[
            pl.BlockSpec((_TB1, _F * _E), lambda i: (i, 0)),
            pl.BlockSpec((_F * _E, _NIP), lambda i: (0, 0)),
            pl.BlockSpec((_TB1, _NIP), lambda i: (i, 0)),
            pl.BlockSpec((_TB1, _NIP), lambda i: (i, 0)),
        ],
        out_specs=[
            pl.BlockSpec((_TB1, _NIP), lambda i: (i, 0)),
            pl.BlockSpec((_TB1, _NIP), lambda i: (i, 0)),
            pl.BlockSpec((_TB1, _NIP), lambda i: (i, 0)),
        ],
        out_shape=[
            jax.ShapeDtypeStruct((_B, _NIP), jnp.float32),
            jax.ShapeDtypeStruct((_B, _NIP), jnp.int32),
            jax.ShapeDtypeStruct((_B, _NIP), jnp.int32),
        ],
    )(z, wse_pad, pa2, pb2)


def _sc_gather_agg(bkt0, bkt1, w_flat, codebook):
    mesh = plsc.VectorSubcoreMesh(core_axis_name="c", subcore_axis_name="s")

    @functools.partial(
        pl.kernel,
        out_type=jax.ShapeDtypeStruct((_B * _ACC_W,), jnp.float32),
        mesh=mesh,
        compiler_params=pltpu.CompilerParams(use_tc_tiling_on_sc=False),
        scratch_types=[
            pltpu.VMEM((_PASS * _ACC_W,), jnp.float32),
            pltpu.VMEM((_PASS * _NSTR, _SW), jnp.int32),
            pltpu.VMEM((_PASS * _NSTR, _SW), jnp.int32),
            pltpu.VMEM((_PASS * _NIP,), jnp.float32),
            pltpu.VMEM((_NIP, _E), jnp.float32),
            pltpu.VMEM((_NIP, _E), jnp.float32),
            pltpu.VMEM((_NIP, _E), jnp.float32),
            pltpu.VMEM((_NIP, _E), jnp.float32),
            pltpu.SemaphoreType.DMA,
            pltpu.SemaphoreType.DMA,
        ],
    )
    def k(b0_hbm, b1_hbm, w_hbm, cb_hbm, acc_hbm,
          acc_v, idx0_v, idx1_v, w_v, r0a_v, r1a_v, r0b_v, r1b_v,
          sema, semb):
        wid = lax.axis_index("s") * _NC + lax.axis_index("c")

        def fire(r, r0_d, r1_d, sem):
            for st in range(_NSTR):
                pltpu.async_copy(cb_hbm.at[idx0_v.at[r * _NSTR + st]],
                                 r0_d.at[pl.ds(st * _SW, _SW)], sem)
                pltpu.async_copy(cb_hbm.at[idx1_v.at[r * _NSTR + st]],
                                 r1_d.at[pl.ds(st * _SW, _SW)], sem)

        def drain(r, r0_d, r1_d, sem):
            for st in range(_NSTR):
                pltpu.make_async_copy(
                    cb_hbm.at[idx0_v.at[r * _NSTR + st]],
                    r0_d.at[pl.ds(st * _SW, _SW)], sem).wait()
                pltpu.make_async_copy(
                    cb_hbm.at[idx1_v.at[r * _NSTR + st]],
                    r1_d.at[pl.ds(st * _SW, _SW)], sem).wait()

        def accum(r, r0_d, r1_d):
            for v in range(_NIP // 16):
                wv = w_v[pl.ds(r * _NIP + v * 16, 16)]
                for sl in range(16):
                    s = v * 16 + sl
                    ws = jnp.broadcast_to(wv[sl], (16,))
                    wr0 = r0_d[s] * ws
                    wr1 = r1_d[s] * ws
                    ci = r * _ACC_W + _FI[s] * 2 * _E
                    cj = r * _ACC_W + _FJ[s] * 2 * _E
                    plsc.addupdate(acc_v.at[pl.ds(ci, 16)], wr0)
                    plsc.addupdate(acc_v.at[pl.ds(ci + 16, 16)], wr1)
                    plsc.addupdate(acc_v.at[pl.ds(cj, 16)], wr0)
                    plsc.addupdate(acc_v.at[pl.ds(cj + 16, 16)], wr1)

        def one_pass(p, carry):
            base_row = wid * _RPW + p * _PASS

            def zero(i, c2):
                acc_v[pl.ds(i * 16, 16)] = jnp.zeros((16,), jnp.float32)
                return c2
            lax.fori_loop(0, _PASS * _ACC_W // 16, zero, 0)

            pltpu.sync_copy(
                b0_hbm.at[pl.ds(base_row * _NSTR, _PASS * _NSTR)], idx0_v)
            pltpu.sync_copy(
                b1_hbm.at[pl.ds(base_row * _NSTR, _PASS * _NSTR)], idx1_v)
            pltpu.sync_copy(
                w_hbm.at[pl.ds(base_row * _NIP, _PASS * _NIP)], w_v)

            fire(0, r0a_v, r1a_v, sema)

            def pair(i, c2):
                r0 = 2 * i
                fire(r0 + 1, r0b_v, r1b_v, semb)
                drain(r0, r0a_v, r1a_v, sema)
                accum(r0, r0a_v, r1a_v)

                @pl.when(i < _PASS // 2 - 1)
                def _():
                    fire(r0 + 2, r0a_v, r1a_v, sema)
                drain(r0 + 1, r0b_v, r1b_v, semb)
                accum(r0 + 1, r0b_v, r1b_v)
                return c2
            lax.fori_loop(0, _PASS // 2, pair, 0)

            pltpu.sync_copy(
                acc_v, acc_hbm.at[pl.ds(base_row * _ACC_W, _PASS * _ACC_W)])
            return carry

        lax.fori_loop(0, _RPW // _PASS, one_pass, 0)

    return k(bkt0, bkt1, w_flat, codebook)


_TB2 = 512


def _out_body(a_ref, wbd_ref, o_ref):
    o_ref[...] = jnp.dot(a_ref[...], wbd_ref[...],
                         preferred_element_type=jnp.float32)


def _out_matmul(acc2, wbd):
    return pl.pallas_call(
        _out_body,
        grid=(_B // _TB2,),
        in_specs=[
            pl.BlockSpec((_TB2, _ACC_W), lambda i: (i, 0)),
            pl.BlockSpec((_ACC_W, _ACC_W), lambda i: (0, 0)),
        ],
        out_specs=pl.BlockSpec((_TB2, _ACC_W), lambda i: (i, 0)),
        out_shape=jax.ShapeDtypeStruct((_B, _ACC_W), jnp.float32),
    )(acc2, wbd)


def kernel(placeholder_inputs, origin_embeddings, codebook, W_t, W_se):
    ids = placeholder_inputs
    pa2 = jnp.take(ids, jnp.asarray(_PA_IDX), axis=1)
    pb2 = jnp.take(ids, jnp.asarray(_PB_IDX), axis=1)
    z = origin_embeddings.reshape(_B, _F * _E)
    wse_pad = jnp.pad(W_se, ((0, 0), (0, _NIP - _NI)))
    w, bkt0, bkt1 = _w_and_hash(z, wse_pad, pa2, pb2)
    cb_lin = _cb_relayout(codebook.T).reshape(_NBKT_PAD, _E)
    acc = _sc_gather_agg(bkt0.reshape(_B * _NSTR, _SW),
                         bkt1.reshape(_B * _NSTR, _SW),
                         w.reshape(-1), cb_lin).reshape(_B, _ACC_W)
    wbd = jnp.kron(jnp.eye(_F, dtype=jnp.float32), W_t)
    out = _out_matmul(acc, wbd)
    return out.reshape(_B, _F, _OUT)


# speedup vs baseline: 35.0815x; 1.0082x over previous
import functools

import numpy as np
import jax
import jax.numpy as jnp
from jax import lax
from jax.experimental import pallas as pl
from jax.experimental.pallas import tpu as pltpu
from jax.experimental.pallas import tpu_sc as plsc

_F = 26
_TOP = 6
_E = 16
_OUT = 32
_B = 4096
_NBKT = 1000000

_PAIR_LIST = [(i, j) for i in range(_F) for j in range(i + 1, _F)
              if (i < _TOP or j < _TOP)]
_NI = len(_PAIR_LIST)
_NIP = 144
_FI = [p[0] for p in _PAIR_LIST] + [0] * (_NIP - _NI)
_FJ = [p[1] for p in _PAIR_LIST] + [0] * (_NIP - _NI)
_PA_IDX = np.array(_FI, dtype=np.int32)
_PB_IDX = np.array(_FJ, dtype=np.int32)

_ACC_W = _F * 2 * _E

_NC = 2
_NS = 16
_NW = _NC * _NS
_RPW = _B // _NW
_PASS = 64
_NSTR = 3
_SW = _NIP // _NSTR

_TCH = 15360
_NBLK = -(-_NBKT // _TCH)
_NBKT_PAD = _NBLK * _TCH


def _relayout_body(ct_ref, o_ref):
    xt = jnp.dot(ct_ref[...].T, jnp.eye(_E, dtype=jnp.float32),
                 preferred_element_type=jnp.float32)
    xt3 = xt.reshape(_TCH // 8, 8, _E)
    for k in range(8):
        o_ref[:, k * _E:(k + 1) * _E] = xt3[:, k, :]


def _cb_relayout(cb_t):
    return pl.pallas_call(
        _relayout_body,
        grid=(_NBLK,),
        in_specs=[pl.BlockSpec((_E, _TCH), lambda i: (0, i))],
        out_specs=pl.BlockSpec((_TCH * _E // 128, 128), lambda i: (i, 0)),
        out_shape=jax.ShapeDtypeStruct((_NBKT_PAD * _E // 128, 128),
                                       jnp.float32),
        compiler_params=pltpu.CompilerParams(
            fuse_transposed_lhs_in_matmul=True),
    )(cb_t)


_TB1 = 512


def _wh_body(z_ref, wse_ref, pa_ref, pb_ref, w_ref, bk_ref):
    w_ref[...] = jnp.dot(z_ref[...], wse_ref[...],
                         preferred_element_type=jnp.float32)
    a = pa_ref[...].astype(jnp.uint32)
    b = pb_ref[...].astype(jnp.uint32)
    tok = a * jnp.uint32(2654435761) + b
    bk_ref[:, :_NIP] = ((tok * jnp.uint32(7744) + jnp.uint32(1822))
                        % jnp.uint32(_NBKT)).astype(jnp.int32)
    bk_ref[:, _NIP:] = ((tok * jnp.uint32(423) + jnp.uint32(6649))
                        % jnp.uint32(_NBKT)).astype(jnp.int32)


def _w_and_hash(z, wse_pad, pa2, pb2):
    return pl.pallas_call(
        _wh_body,
        grid=(_B // _TB1,),
        in_specs=[
            pl.BlockSpec((_TB1, _F * _E), lambda i: (i, 0)),
            pl.BlockSpec((_F * _E, _NIP), lambda i: (0, 0)),
            pl.BlockSpec((_TB1, _NIP), lambda i: (i, 0)),
            pl.BlockSpec((_TB1, _NIP), lambda i: (i, 0)),
        ],
        out_specs=[
            pl.BlockSpec((_TB1, _NIP), lambda i: (i, 0)),
            pl.BlockSpec((_TB1, 2 * _NIP), lambda i: (i, 0)),
        ],
        out_shape=[
            jax.ShapeDtypeStruct((_B, _NIP), jnp.float32),
            jax.ShapeDtypeStruct((_B, 2 * _NIP), jnp.int32),
        ],
    )(z, wse_pad, pa2, pb2)


_RNI = 2 * _NIP


def _sc_gather_agg(bkt, w_flat, codebook):
    mesh = plsc.VectorSubcoreMesh(core_axis_name="c", subcore_axis_name="s")

    @functools.partial(
        pl.kernel,
        out_type=jax.ShapeDtypeStruct((_B * _ACC_W,), jnp.float32),
        mesh=mesh,
        compiler_params=pltpu.CompilerParams(use_tc_tiling_on_sc=False),
        scratch_types=[
            pltpu.VMEM((_PASS * _ACC_W,), jnp.float32),
            pltpu.VMEM((_PASS * _RNI,), jnp.int32),
            pltpu.VMEM((_PASS * _NIP,), jnp.float32),
            pltpu.VMEM((_RNI, _E), jnp.float32),
            pltpu.VMEM((_RNI, _E), jnp.float32),
            pltpu.SemaphoreType.DMA,
            pltpu.SemaphoreType.DMA,
        ],
    )
    def k(bk_hbm, w_hbm, cb_hbm, acc_hbm,
          acc_v, idx_v, w_v, ra_v, rb_v, sema, semb):
        wid = lax.axis_index("s") * _NC + lax.axis_index("c")

        def streams(r, dst, sem):
            base = r * _RNI
            return [
                (cb_hbm.at[idx_v.at[pl.ds(base, 128)]],
                 dst.at[pl.ds(0, 128)], sem),
                (cb_hbm.at[idx_v.at[pl.ds(base + 128, 128)]],
                 dst.at[pl.ds(128, 128)], sem),
                (cb_hbm.at[idx_v.at[pl.ds(base + 256, 32)]],
                 dst.at[pl.ds(256, 32)], sem),
            ]

        def fire(r, dst, sem):
            for src_s, dst_s, sem_s in streams(r, dst, sem):
                pltpu.async_copy(src_s, dst_s, sem_s)

        def drain(r, dst, sem):
            for src_s, dst_s, sem_s in streams(r, dst, sem):
                pltpu.make_async_copy(src_s, dst_s, sem_s).wait()

        def accum(r, r_d):
            for v in range(_NIP // 16):
                wv = w_v[pl.ds(r * _NIP + v * 16, 16)]
                for sl in range(16):
                    s = v * 16 + sl
                    ws = jnp.broadcast_to(wv[sl], (16,))
                    wr0 = r_d[s] * ws
                    wr1 = r_d[_NIP + s] * ws
                    ci = r * _ACC_W + _FI[s] * 2 * _E
                    cj = r * _ACC_W + _FJ[s] * 2 * _E
                    plsc.addupdate(acc_v.at[pl.ds(ci, 16)], wr0)
                    plsc.addupdate(acc_v.at[pl.ds(ci + 16, 16)], wr1)
                    plsc.addupdate(acc_v.at[pl.ds(cj, 16)], wr0)
                    plsc.addupdate(acc_v.at[pl.ds(cj + 16, 16)], wr1)

        def one_pass(p, carry):
            base_row = wid * _RPW + p * _PASS

            def zero(i, c2):
                acc_v[pl.ds(i * 16, 16)] = jnp.zeros((16,), jnp.float32)
                return c2
            lax.fori_loop(0, _PASS * _ACC_W // 16, zero, 0)

            pltpu.sync_copy(
                bk_hbm.at[pl.ds(base_row * _RNI, _PASS * _RNI)], idx_v)
            pltpu.sync_copy(
                w_hbm.at[pl.ds(base_row * _NIP, _PASS * _NIP)], w_v)

            fire(0, ra_v, sema)

            def pair(i, c2):
                r0 = 2 * i
                fire(r0 + 1, rb_v, semb)
                drain(r0, ra_v, sema)
                accum(r0, ra_v)

                @pl.when(i < _PASS // 2 - 1)
                def _():
                    fire(r0 + 2, ra_v, sema)
                drain(r0 + 1, rb_v, semb)
                accum(r0 + 1, rb_v)
                return c2
            lax.fori_loop(0, _PASS // 2, pair, 0)

            pltpu.sync_copy(
                acc_v, acc_hbm.at[pl.ds(base_row * _ACC_W, _PASS * _ACC_W)])
            return carry

        lax.fori_loop(0, _RPW // _PASS, one_pass, 0)

    return k(bkt, w_flat, codebook)


_TB2 = 512


def _out_body(a_ref, wbd_ref, o_ref):
    o_ref[...] = jnp.dot(a_ref[...], wbd_ref[...],
                         preferred_element_type=jnp.float32)


def _out_matmul(acc2, wbd):
    return pl.pallas_call(
        _out_body,
        grid=(_B // _TB2,),
        in_specs=[
            pl.BlockSpec((_TB2, _ACC_W), lambda i: (i, 0)),
            pl.BlockSpec((_ACC_W, _ACC_W), lambda i: (0, 0)),
        ],
        out_specs=pl.BlockSpec((_TB2, _ACC_W), lambda i: (i, 0)),
        out_shape=jax.ShapeDtypeStruct((_B, _ACC_W), jnp.float32),
    )(acc2, wbd)


def kernel(placeholder_inputs, origin_embeddings, codebook, W_t, W_se):
    ids = placeholder_inputs
    pa2 = jnp.take(ids, jnp.asarray(_PA_IDX), axis=1)
    pb2 = jnp.take(ids, jnp.asarray(_PB_IDX), axis=1)
    z = origin_embeddings.reshape(_B, _F * _E)
    wse_pad = jnp.pad(W_se, ((0, 0), (0, _NIP - _NI)))
    w, bkt = _w_and_hash(z, wse_pad, pa2, pb2)
    cb_lin = _cb_relayout(codebook.T).reshape(_NBKT_PAD, _E)
    acc = _sc_gather_agg(bkt.reshape(-1), w.reshape(-1),
                         cb_lin).reshape(_B, _ACC_W)
    wbd = jnp.kron(jnp.eye(_F, dtype=jnp.float32), W_t)
    out = _out_matmul(acc, wbd)
    return out.reshape(_B, _F, _OUT)


# speedup vs baseline: 35.2941x; 1.0061x over previous
import functools

import numpy as np
import jax
import jax.numpy as jnp
from jax import lax
from jax.experimental import pallas as pl
from jax.experimental.pallas import tpu as pltpu
from jax.experimental.pallas import tpu_sc as plsc

_F = 26
_TOP = 6
_E = 16
_OUT = 32
_B = 4096
_NBKT = 1000000

_PAIR_LIST = [(i, j) for i in range(_F) for j in range(i + 1, _F)
              if (i < _TOP or j < _TOP)]
_NI = len(_PAIR_LIST)
_NIP = 144
_FI = [p[0] for p in _PAIR_LIST] + [0] * (_NIP - _NI)
_FJ = [p[1] for p in _PAIR_LIST] + [0] * (_NIP - _NI)
_PA_IDX = np.array(_FI, dtype=np.int32)
_PB_IDX = np.array(_FJ, dtype=np.int32)

_ACC_W = _F * 2 * _E

_NC = 2
_NS = 16
_NW = _NC * _NS
_RPW = _B // _NW
_PASS = 64

_TCH = 15360
_NBLK = -(-_NBKT // _TCH)
_NBKT_PAD = _NBLK * _TCH


def _relayout_body(ct_ref, o_ref):
    xt = jnp.dot(ct_ref[...].T, jnp.eye(_E, dtype=jnp.float32),
                 preferred_element_type=jnp.float32)
    xt3 = xt.reshape(_TCH // 8, 8, _E)
    for k in range(8):
        o_ref[:, k * _E:(k + 1) * _E] = xt3[:, k, :]


def _cb_relayout(cb_t):
    return pl.pallas_call(
        _relayout_body,
        grid=(_NBLK,),
        in_specs=[pl.BlockSpec((_E, _TCH), lambda i: (0, i))],
        out_specs=pl.BlockSpec((_TCH * _E // 128, 128), lambda i: (i, 0)),
        out_shape=jax.ShapeDtypeStruct((_NBKT_PAD * _E // 128, 128),
                                       jnp.float32),
        compiler_params=pltpu.CompilerParams(
            fuse_transposed_lhs_in_matmul=True),
    )(cb_t)


_TB1 = 512


def _wh_body(z_ref, wse_ref, pa_ref, pb_ref, w_ref, bk_ref):
    w_ref[...] = jnp.dot(z_ref[...], wse_ref[...],
                         preferred_element_type=jnp.float32)
    a = pa_ref[...].astype(jnp.uint32)
    b = pb_ref[...].astype(jnp.uint32)
    tok = a * jnp.uint32(2654435761) + b
    bk_ref[:, :_NIP] = ((tok * jnp.uint32(7744) + jnp.uint32(1822))
                        % jnp.uint32(_NBKT)).astype(jnp.int32)
    bk_ref[:, _NIP:] = ((tok * jnp.uint32(423) + jnp.uint32(6649))
                        % jnp.uint32(_NBKT)).astype(jnp.int32)


def _w_and_hash(z, wse_pad, pa2, pb2):
    return pl.pallas_call(
        _wh_body,
        grid=(_B // _TB1,),
        in_specs=[
            pl.BlockSpec((_TB1, _F * _E), lambda i: (i, 0)),
            pl.BlockSpec((_F * _E, _NIP), lambda i: (0, 0)),
            pl.BlockSpec((_TB1, _NIP), lambda i: (i, 0)),
            pl.BlockSpec((_TB1, _NIP), lambda i: (i, 0)),
        ],
        out_specs=[
            pl.BlockSpec((_TB1, _NIP), lambda i: (i, 0)),
            pl.BlockSpec((_TB1, 2 * _NIP), lambda i: (i, 0)),
        ],
        out_shape=[
            jax.ShapeDtypeStruct((_B, _NIP), jnp.float32),
            jax.ShapeDtypeStruct((_B, 2 * _NIP), jnp.int32),
        ],
    )(z, wse_pad, pa2, pb2)


_RNI = 2 * _NIP


def _sc_gather_agg(bkt, w_flat, codebook):
    mesh = plsc.VectorSubcoreMesh(core_axis_name="c", subcore_axis_name="s")

    @functools.partial(
        pl.kernel,
        out_type=jax.ShapeDtypeStruct((_B * _ACC_W,), jnp.float32),
        mesh=mesh,
        compiler_params=pltpu.CompilerParams(use_tc_tiling_on_sc=False),
        scratch_types=[
            pltpu.VMEM((_PASS * _ACC_W,), jnp.float32),
            pltpu.VMEM((_PASS * _RNI,), jnp.int32),
            pltpu.VMEM((_PASS * _NIP,), jnp.float32),
            pltpu.VMEM((_RNI, _E), jnp.float32),
            pltpu.VMEM((_RNI, _E), jnp.float32),
            pltpu.SemaphoreType.DMA,
            pltpu.SemaphoreType.DMA,
        ],
    )
    def k(bk_hbm, w_hbm, cb_hbm, acc_hbm,
          acc_v, idx_v, w_v, ra_v, rb_v, sema, semb):
        wid = lax.axis_index("s") * _NC + lax.axis_index("c")

        def streams(r, dst, sem):
            base = r * _RNI
            return [
                (cb_hbm.at[idx_v.at[pl.ds(base, 128)]],
                 dst.at[pl.ds(0, 128)], sem),
                (cb_hbm.at[idx_v.at[pl.ds(base + 128, 128)]],
                 dst.at[pl.ds(128, 128)], sem),
                (cb_hbm.at[idx_v.at[pl.ds(base + 256, 32)]],
                 dst.at[pl.ds(256, 32)], sem),
            ]

        def fire(r, dst, sem):
            for src_s, dst_s, sem_s in streams(r, dst, sem):
                pltpu.async_copy(src_s, dst_s, sem_s)

        def drain(r, dst, sem):
            for src_s, dst_s, sem_s in streams(r, dst, sem):
                pltpu.make_async_copy(src_s, dst_s, sem_s).wait()

        def accum(r, r_d):
            for v in range(_NIP // 16):
                wv = w_v[pl.ds(r * _NIP + v * 16, 16)]
                for sl in range(16):
                    s = v * 16 + sl
                    ws = jnp.broadcast_to(wv[sl], (16,))
                    wr0 = r_d[s] * ws
                    wr1 = r_d[_NIP + s] * ws
                    ci = r * _ACC_W + _FI[s] * 2 * _E
                    cj = r * _ACC_W + _FJ[s] * 2 * _E
                    plsc.addupdate(acc_v.at[pl.ds(ci, 16)], wr0)
                    plsc.addupdate(acc_v.at[pl.ds(ci + 16, 16)], wr1)
                    plsc.addupdate(acc_v.at[pl.ds(cj, 16)], wr0)
                    plsc.addupdate(acc_v.at[pl.ds(cj + 16, 16)], wr1)

        def one_pass(p, carry):
            base_row = wid * _RPW + p * _PASS

            def zero(i, c2):
                acc_v[pl.ds(i * 16, 16)] = jnp.zeros((16,), jnp.float32)
                return c2
            lax.fori_loop(0, _PASS * _ACC_W // 16, zero, 0)

            pltpu.sync_copy(
                bk_hbm.at[pl.ds(base_row * _RNI, _PASS * _RNI)], idx_v)
            pltpu.sync_copy(
                w_hbm.at[pl.ds(base_row * _NIP, _PASS * _NIP)], w_v)

            fire(0, ra_v, sema)

            def pair(i, c2):
                r0 = 2 * i
                fire(r0 + 1, rb_v, semb)
                drain(r0, ra_v, sema)
                accum(r0, ra_v)

                @pl.when(i < _PASS // 2 - 1)
                def _():
                    fire(r0 + 2, ra_v, sema)
                drain(r0 + 1, rb_v, semb)
                accum(r0 + 1, rb_v)
                return c2
            lax.fori_loop(0, _PASS // 2, pair, 0)

            pltpu.sync_copy(
                acc_v, acc_hbm.at[pl.ds(base_row * _ACC_W, _PASS * _ACC_W)])
            return carry

        lax.fori_loop(0, _RPW // _PASS, one_pass, 0)

    return k(bkt, w_flat, codebook)


_TB2 = 512


def _out_body(a_ref, wbd_ref, o_ref):
    o_ref[...] = jnp.dot(a_ref[...], wbd_ref[...],
                         preferred_element_type=jnp.float32)


def _out_matmul(acc2, wbd):
    return pl.pallas_call(
        _out_body,
        grid=(_B // _TB2,),
        in_specs=[
            pl.BlockSpec((_TB2, _ACC_W), lambda i: (i, 0)),
            pl.BlockSpec((_ACC_W, _ACC_W), lambda i: (0, 0)),
        ],
        out_specs=pl.BlockSpec((_TB2, _ACC_W), lambda i: (i, 0)),
        out_shape=jax.ShapeDtypeStruct((_B, _ACC_W), jnp.float32),
    )(acc2, wbd)


def kernel(placeholder_inputs, origin_embeddings, codebook, W_t, W_se):
    ids = placeholder_inputs
    pa2 = jnp.take(ids, jnp.asarray(_PA_IDX), axis=1)
    pb2 = jnp.take(ids, jnp.asarray(_PB_IDX), axis=1)
    z = origin_embeddings.reshape(_B, _F * _E)
    wse_pad = jnp.pad(W_se, ((0, 0), (0, _NIP - _NI)))
    w, bkt = _w_and_hash(z, wse_pad, pa2, pb2)
    cb_lin = _cb_relayout(codebook.T).reshape(_NBKT_PAD, _E)
    acc = _sc_gather_agg(bkt.reshape(-1), w.reshape(-1),
                         cb_lin).reshape(_B, _ACC_W)
    wbd = jnp.kron(jnp.eye(_F, dtype=jnp.float32), W_t)
    out = _out_matmul(acc, wbd)
    return out.reshape(_B, _F, _OUT)
